# Initial kernel scaffold; baseline (speedup 1.0000x reference)
import jax
import jax.numpy as jnp
from jax.experimental import pallas as pl


def kernel(x, edge_index, batch, W1, att_src1, att_dst1, b1, W2, att_src2, att_dst2, b2, fc_W, fc_b):
    raise NotImplementedError("write your pallas kernel here")


# speedup vs baseline: 186.7453x; 186.7453x over previous
import functools

import jax
import jax.numpy as jnp
from jax import lax
from jax.experimental import pallas as pl
from jax.experimental.pallas import tpu as pltpu
from jax.experimental.pallas import tpu_sc as plsc

N = 100000
NPAD = 100096
PAD = NPAD - N
E = 6400000
G = 128
H = 8
L = 16
NC = 2
NS = 16
NW = NC * NS
EPW = E // NW
C = 128
NFULL = EPW // C
TAIL = EPW - NFULL * C
ZR = NPAD // NS
ZC = 136

_mesh = plsc.VectorSubcoreMesh(
    core_axis_name="c", subcore_axis_name="s", num_cores=NC, num_subcores=NS)
_sc_params = pltpu.CompilerParams(
    needs_layout_passes=False, use_tc_tiling_on_sc=False)


def _leaky_exp(a):
    return jnp.exp(jnp.where(a > 0, a, 0.2 * a))


def _fill_zero(zbuf):
    zv = jnp.zeros((L,), jnp.float32)
    iota0 = lax.iota(jnp.int32, L)
    for k in range(ZC * 16 // L):
        flat = iota0 + k * L
        plsc.store_scatter(
            zbuf, [lax.shift_right_logical(flat, 4),
                   lax.bitwise_and(flat, 15)], zv)


@functools.partial(
    pl.kernel,
    out_type=jax.ShapeDtypeStruct((NC, NPAD, 16), jnp.float32),
    mesh=_mesh,
    compiler_params=_sc_params,
    scratch_types=[
        pltpu.VMEM((16, 16), jnp.float32),
        pltpu.VMEM((C,), jnp.int32),
        pltpu.VMEM((C,), jnp.int32),
        pltpu.VMEM((C,), jnp.float32),
        pltpu.VMEM((C,), jnp.float32),
        pltpu.VMEM((C, 16), jnp.float32),
        pltpu.VMEM((TAIL,), jnp.int32),
        pltpu.VMEM((TAIL,), jnp.int32),
        pltpu.VMEM((TAIL,), jnp.float32),
        pltpu.VMEM((TAIL,), jnp.float32),
        pltpu.VMEM((TAIL, 16), jnp.float32),
        pltpu.VMEM((ZC, 16), jnp.float32),
        pltpu.VMEM_SHARED((N,), jnp.float32),
        pltpu.VMEM_SHARED((NPAD, 16), jnp.float32),
        pltpu.SemaphoreType.DMA,
    ],
)
def _edge_pass1(x_hbm, src_hbm, dst_hbm, coef_hbm, out_hbm,
                coef_v, src_v, dst_v, xs_v, xd_v, stag_v,
                src_t, dst_t, xs_t, xd_t, stag_t, zbuf, x_sh, acc_sh, sem):
    cid = lax.axis_index("c")
    sid = lax.axis_index("s")
    wid = cid * NS + sid

    _fill_zero(zbuf)

    def zero_body(k, carry):
        pltpu.sync_copy(zbuf, acc_sh.at[pl.ds(sid * ZR + k * ZC, ZC)])
        return carry
    lax.fori_loop(0, ZR // ZC, zero_body, 0)

    @pl.when(sid == 0)
    def _load_x():
        pltpu.sync_copy(x_hbm, x_sh)

    pltpu.sync_copy(coef_hbm, coef_v)
    s1 = [coef_v[h] for h in range(H)]
    d1 = [coef_v[H + h] for h in range(H)]
    plsc.subcore_barrier()

    ebase = wid * EPW
    iota = lax.iota(jnp.int32, L)

    def chunk(off, sbuf, dbuf, xsb, xdb, stbuf, csz):
        pltpu.sync_copy(src_hbm.at[pl.ds(off, csz)], sbuf)
        pltpu.sync_copy(dst_hbm.at[pl.ds(off, csz)], dbuf)
        pltpu.async_copy(x_sh.at[sbuf], xsb, sem).wait()
        pltpu.async_copy(x_sh.at[dbuf], xdb, sem).wait()
        for v in range(csz // L):
            xs = xsb[pl.ds(v * L, L)]
            xd = xdb[pl.ds(v * L, L)]
            row = iota + (v * L)
            for h in range(H):
                p = _leaky_exp(xs * s1[h] + xd * d1[h])
                plsc.store_scatter(
                    stbuf, [row, jnp.full((L,), h, jnp.int32)], xs * p)
                plsc.store_scatter(
                    stbuf, [row, jnp.full((L,), H + h, jnp.int32)], p)
        pltpu.sync_copy(stbuf, acc_sh.at[dbuf], add=True)

    def main_body(i, carry):
        chunk(ebase + i * C, src_v, dst_v, xs_v, xd_v, stag_v, C)
        return carry
    lax.fori_loop(0, NFULL, main_body, 0)
    chunk(ebase + NFULL * C, src_t, dst_t, xs_t, xd_t, stag_t, TAIL)

    plsc.subcore_barrier()
    pltpu.sync_copy(acc_sh.at[pl.ds(sid * ZR, ZR)],
                    out_hbm.at[cid, pl.ds(sid * ZR, ZR)])


@functools.partial(
    pl.kernel,
    out_type=jax.ShapeDtypeStruct((NC, NPAD, 16), jnp.float32),
    mesh=_mesh,
    compiler_params=_sc_params,
    scratch_types=[
        pltpu.VMEM((C,), jnp.int32),
        pltpu.VMEM((C,), jnp.int32),
        pltpu.VMEM((C, 16), jnp.float32),
        pltpu.VMEM((C,), jnp.float32),
        pltpu.VMEM((C, 16), jnp.float32),
        pltpu.VMEM((TAIL,), jnp.int32),
        pltpu.VMEM((TAIL,), jnp.int32),
        pltpu.VMEM((TAIL, 16), jnp.float32),
        pltpu.VMEM((TAIL,), jnp.float32),
        pltpu.VMEM((TAIL, 16), jnp.float32),
        pltpu.VMEM((ZC, 16), jnp.float32),
        pltpu.VMEM_SHARED((NPAD,), jnp.float32),
        pltpu.VMEM_SHARED((NPAD, 16), jnp.float32),
        pltpu.SemaphoreType.DMA,
    ],
)
def _edge_pass2(hrec_hbm, a2d_hbm, src_hbm, dst_hbm, out_hbm,
                src_v, dst_v, gbuf, ad_v, stag_v,
                src_t, dst_t, gbuf_t, ad_t, stag_t, zbuf, a2d_sh, acc_sh,
                sem):
    cid = lax.axis_index("c")
    sid = lax.axis_index("s")
    wid = cid * NS + sid

    _fill_zero(zbuf)

    def zero_body(k, carry):
        pltpu.sync_copy(zbuf, acc_sh.at[pl.ds(sid * ZR + k * ZC, ZC)])
        return carry
    lax.fori_loop(0, ZR // ZC, zero_body, 0)

    @pl.when(sid == 0)
    def _load_a2d():
        pltpu.sync_copy(a2d_hbm, a2d_sh)

    plsc.subcore_barrier()

    ebase = wid * EPW
    iota = lax.iota(jnp.int32, L)
    col8 = jnp.full((L,), 8, jnp.int32)

    def chunk(off, sbuf, dbuf, gb, adb, stbuf, csz):
        pltpu.sync_copy(src_hbm.at[pl.ds(off, csz)], sbuf)
        pltpu.sync_copy(dst_hbm.at[pl.ds(off, csz)], dbuf)
        pltpu.async_copy(hrec_hbm.at[sbuf], gb, sem).wait()
        pltpu.async_copy(a2d_sh.at[dbuf], adb, sem).wait()
        for v in range(csz // L):
            ad = adb[pl.ds(v * L, L)]
            row = iota + (v * L)
            a2s = plsc.load_gather(gb, [row, col8])
            p = _leaky_exp(a2s + ad)
            plsc.store_scatter(stbuf, [row, col8], p)
            for j in range(8):
                cj = jnp.full((L,), j, jnp.int32)
                hj = plsc.load_gather(gb, [row, cj])
                plsc.store_scatter(stbuf, [row, cj], hj * p)
        pltpu.sync_copy(stbuf, acc_sh.at[dbuf], add=True)

    def main_body(i, carry):
        chunk(ebase + i * C, src_v, dst_v, gbuf, ad_v, stag_v, C)
        return carry
    lax.fori_loop(0, NFULL, main_body, 0)
    chunk(ebase + NFULL * C, src_t, dst_t, gbuf_t, ad_t, stag_t, TAIL)

    plsc.subcore_barrier()
    pltpu.sync_copy(acc_sh.at[pl.ds(sid * ZR, ZR)],
                    out_hbm.at[cid, pl.ds(sid * ZR, ZR)])


NB_A = 6256


def _tc_a_body(part_ref, w1_ref, b1_ref, w2_ref, as2_ref, ad2_ref,
               hrec_ref, a2d_ref):
    num = part_ref[0, :, 0:8] + part_ref[1, :, 0:8]
    den = part_ref[0, :, 8:16] + part_ref[1, :, 8:16]
    t = num / (den + 1e-16)
    ki = lax.broadcasted_iota(jnp.int32, (8, 64), 1) // 8
    hi = lax.broadcasted_iota(jnp.int32, (8, 64), 0)
    rep = (ki == hi).astype(jnp.float32)
    trep = jnp.dot(t, rep, preferred_element_type=jnp.float32)
    h1 = trep * w1_ref[:] + b1_ref[:]
    h1 = jnp.where(h1 > 0, h1, jnp.exp(h1) - 1.0)
    h2 = jnp.dot(h1, w2_ref[:], preferred_element_type=jnp.float32)
    a2s = jnp.sum(h2 * as2_ref[:], axis=1, keepdims=True)
    a2d = jnp.sum(h2 * ad2_ref[:], axis=1, keepdims=True)
    hrec_ref[:, 0:8] = h2
    hrec_ref[:, 8:9] = a2s
    hrec_ref[:, 9:16] = jnp.zeros((h2.shape[0], 7), jnp.float32)
    a2d_ref[:, :] = a2d


def _tc_a(part1, W1, b1, W2, as2, ad2):
    return pl.pallas_call(
        _tc_a_body,
        grid=(NPAD // NB_A,),
        in_specs=[
            pl.BlockSpec((NC, NB_A, 16), lambda i: (0, i, 0)),
            pl.BlockSpec((1, 64), lambda i: (0, 0)),
            pl.BlockSpec((1, 64), lambda i: (0, 0)),
            pl.BlockSpec((64, 8), lambda i: (0, 0)),
            pl.BlockSpec((1, 8), lambda i: (0, 0)),
            pl.BlockSpec((1, 8), lambda i: (0, 0)),
        ],
        out_specs=[
            pl.BlockSpec((NB_A, 16), lambda i: (i, 0)),
            pl.BlockSpec((NB_A, 1), lambda i: (i, 0)),
        ],
        out_shape=[
            jax.ShapeDtypeStruct((NPAD, 16), jnp.float32),
            jax.ShapeDtypeStruct((NPAD, 1), jnp.float32),
        ],
    )(part1, W1, b1, W2, as2, ad2)


NB_B = 6256


def _tc_b_body(part_ref, batch_ref, b2_ref, fcw_ref, fcb_ref, out_ref,
               sums, counts):
    i = pl.program_id(0)
    ng = pl.num_programs(0)

    @pl.when(i == 0)
    def _init():
        sums[:, :] = jnp.zeros((G, 8), jnp.float32)
        counts[:, :] = jnp.zeros((G, 1), jnp.float32)

    num = part_ref[0, :, 0:8] + part_ref[1, :, 0:8]
    den = part_ref[0, :, 8:9] + part_ref[1, :, 8:9]
    out2 = num / (den + 1e-16) + b2_ref[:]
    bb = batch_ref[:]
    gi = lax.broadcasted_iota(jnp.int32, (bb.shape[0], G), 1)
    onehot = (bb == gi).astype(jnp.float32)
    sums[:, :] += lax.dot_general(
        onehot, out2, (((0,), (0,)), ((), ())),
        preferred_element_type=jnp.float32)
    counts[:, :] += lax.dot_general(
        onehot, jnp.ones((bb.shape[0], 1), jnp.float32),
        (((0,), (0,)), ((), ())), preferred_element_type=jnp.float32)

    @pl.when(i == ng - 1)
    def _final():
        pooled = sums[:, :] / jnp.maximum(counts[:, :], 1.0)
        logit = jnp.dot(pooled, fcw_ref[:],
                        preferred_element_type=jnp.float32) + fcb_ref[:]
        out_ref[:] = jnp.squeeze(1.0 / (1.0 + jnp.exp(-logit)), axis=1)


def _tc_b(part2, batch2d, b2, fc_W, fc_b):
    return pl.pallas_call(
        _tc_b_body,
        grid=(NPAD // NB_B,),
        in_specs=[
            pl.BlockSpec((NC, NB_B, 16), lambda i: (0, i, 0)),
            pl.BlockSpec((NB_B, 1), lambda i: (i, 0)),
            pl.BlockSpec((1, 8), lambda i: (0, 0)),
            pl.BlockSpec((8, 1), lambda i: (0, 0)),
            pl.BlockSpec((1, 1), lambda i: (0, 0)),
        ],
        out_specs=pl.BlockSpec((G,), lambda i: (0,)),
        out_shape=jax.ShapeDtypeStruct((G,), jnp.float32),
        scratch_shapes=[
            pltpu.VMEM((G, 8), jnp.float32),
            pltpu.VMEM((G, 1), jnp.float32),
        ],
    )(part2, batch2d, b2, fc_W, fc_b)


def kernel(x, edge_index, batch, W1, att_src1, att_dst1, b1,
           W2, att_src2, att_dst2, b2, fc_W, fc_b):
    src = edge_index[0]
    dst = edge_index[1]
    xf = x.reshape(N)
    W1r = W1.reshape(8, 8)
    s1 = (W1r * att_src1).sum(1)
    d1 = (W1r * att_dst1).sum(1)
    coef = jnp.broadcast_to(
        jnp.concatenate([s1, d1])[:, None], (16, 16)).astype(jnp.float32)
    batch_p = jnp.concatenate(
        [batch, jnp.full((PAD,), G, jnp.int32)]).reshape(NPAD, 1)

    part1 = _edge_pass1(xf, src, dst, coef)
    hrec, a2d = _tc_a(part1, W1, b1.reshape(1, 64), W2, att_src2, att_dst2)
    part2 = _edge_pass2(hrec, a2d.reshape(NPAD), src, dst)
    return _tc_b(part2, batch_p, b2.reshape(1, 8),
                 fc_W, fc_b.reshape(1, 1))


# speedup vs baseline: 249.9982x; 1.3387x over previous
import functools

import jax
import jax.numpy as jnp
from jax import lax
from jax.experimental import pallas as pl
from jax.experimental.pallas import tpu as pltpu
from jax.experimental.pallas import tpu_sc as plsc

N = 100000
NPAD = 100096
PAD = NPAD - N
E = 6400000
G = 128
H = 8
L = 16
NC = 2
NS = 16
NW = NC * NS
EPW = E // NW
C = 128
NFULL = EPW // C
TAIL = EPW - NFULL * C
ZR = NPAD // NS
ZC = 136

_mesh = plsc.VectorSubcoreMesh(
    core_axis_name="c", subcore_axis_name="s", num_cores=NC, num_subcores=NS)
_sc_params = pltpu.CompilerParams(
    needs_layout_passes=False, use_tc_tiling_on_sc=False)


def _leaky_exp(a):
    return jnp.exp(jnp.where(a > 0, a, 0.2 * a))


def _fill_zero(zbuf):
    zv = jnp.zeros((L,), jnp.float32)
    iota0 = lax.iota(jnp.int32, L)
    for k in range(ZC * 16 // L):
        flat = iota0 + k * L
        plsc.store_scatter(
            zbuf, [lax.shift_right_logical(flat, 4),
                   lax.bitwise_and(flat, 15)], zv)


@functools.partial(
    pl.kernel,
    out_type=jax.ShapeDtypeStruct((NC, NPAD, 16), jnp.float32),
    mesh=_mesh,
    compiler_params=_sc_params,
    scratch_types=[
        pltpu.VMEM((16, 16), jnp.float32),
        pltpu.VMEM((C,), jnp.int32),
        pltpu.VMEM((C,), jnp.int32),
        pltpu.VMEM((C,), jnp.float32),
        pltpu.VMEM((C,), jnp.float32),
        pltpu.VMEM((C, 16), jnp.float32),
        pltpu.VMEM((C,), jnp.int32),
        pltpu.VMEM((C,), jnp.int32),
        pltpu.VMEM((C,), jnp.float32),
        pltpu.VMEM((C,), jnp.float32),
        pltpu.VMEM((C, 16), jnp.float32),
        pltpu.VMEM((TAIL,), jnp.int32),
        pltpu.VMEM((TAIL,), jnp.int32),
        pltpu.VMEM((TAIL,), jnp.float32),
        pltpu.VMEM((TAIL,), jnp.float32),
        pltpu.VMEM((TAIL, 16), jnp.float32),
        pltpu.VMEM((ZC, 16), jnp.float32),
        pltpu.VMEM_SHARED((N,), jnp.float32),
        pltpu.VMEM_SHARED((NPAD, 16), jnp.float32),
        pltpu.SemaphoreType.DMA,
        pltpu.SemaphoreType.DMA,
        pltpu.SemaphoreType.DMA,
        pltpu.SemaphoreType.DMA,
        pltpu.SemaphoreType.DMA,
        pltpu.SemaphoreType.DMA,
    ],
)
def _edge_pass1(x_hbm, src_hbm, dst_hbm, coef_hbm, out_hbm,
                coef_v, src_v, dst_v, xs_v, xd_v, stag_v,
                src_b, dst_b, xs_b, xd_b, stag_b,
                src_t, dst_t, xs_t, xd_t, stag_t, zbuf, x_sh, acc_sh, sem,
                semia, semib, semga, semgb, semsc):
    cid = lax.axis_index("c")
    sid = lax.axis_index("s")
    wid = cid * NS + sid

    _fill_zero(zbuf)

    def zero_body(k, carry):
        pltpu.sync_copy(zbuf, acc_sh.at[pl.ds(sid * ZR + k * ZC, ZC)])
        return carry
    lax.fori_loop(0, ZR // ZC, zero_body, 0)

    @pl.when(sid == 0)
    def _load_x():
        pltpu.sync_copy(x_hbm, x_sh)

    pltpu.sync_copy(coef_hbm, coef_v)
    s1 = [coef_v[h] for h in range(H)]
    d1 = [coef_v[H + h] for h in range(H)]
    plsc.subcore_barrier()

    ebase = wid * EPW
    iota = lax.iota(jnp.int32, L)

    def compute(xsb, xdb, stbuf, csz):
        for v in range(csz // L):
            xs = xsb[pl.ds(v * L, L)]
            xd = xdb[pl.ds(v * L, L)]
            row = iota + (v * L)
            for h in range(H):
                p = _leaky_exp(xs * s1[h] + xd * d1[h])
                plsc.store_scatter(
                    stbuf, [row, jnp.full((L,), h, jnp.int32)], xs * p)
                plsc.store_scatter(
                    stbuf, [row, jnp.full((L,), H + h, jnp.int32)], p)

    toff = ebase + NFULL * C
    pltpu.sync_copy(src_hbm.at[pl.ds(toff, TAIL)], src_t)
    pltpu.sync_copy(dst_hbm.at[pl.ds(toff, TAIL)], dst_t)
    pltpu.async_copy(x_sh.at[src_t], xs_t, sem).wait()
    pltpu.async_copy(x_sh.at[dst_t], xd_t, sem).wait()
    compute(xs_t, xd_t, stag_t, TAIL)
    pltpu.sync_copy(stag_t, acc_sh.at[dst_t], add=True)

    def pair_body(k, carry):
        offa = ebase + (2 * k) * C
        offb = offa + C
        dia1 = pltpu.async_copy(src_hbm.at[pl.ds(offa, C)], src_v, semia)
        dia2 = pltpu.async_copy(dst_hbm.at[pl.ds(offa, C)], dst_v, semia)
        dib1 = pltpu.async_copy(src_hbm.at[pl.ds(offb, C)], src_b, semib)
        dib2 = pltpu.async_copy(dst_hbm.at[pl.ds(offb, C)], dst_b, semib)
        dia1.wait()
        dia2.wait()
        pltpu.async_copy(x_sh.at[src_v], xs_v, semga).wait()
        pltpu.async_copy(x_sh.at[dst_v], xd_v, semga).wait()
        compute(xs_v, xd_v, stag_v, C)
        pltpu.sync_copy(stag_v, acc_sh.at[dst_v], add=True)
        dib1.wait()
        dib2.wait()
        pltpu.async_copy(x_sh.at[src_b], xs_b, semgb).wait()
        pltpu.async_copy(x_sh.at[dst_b], xd_b, semgb).wait()
        compute(xs_b, xd_b, stag_b, C)
        pltpu.sync_copy(stag_b, acc_sh.at[dst_b], add=True)
        return carry
    lax.fori_loop(0, NFULL // 2, pair_body, 0)

    plsc.subcore_barrier()
    pltpu.sync_copy(acc_sh.at[pl.ds(sid * ZR, ZR)],
                    out_hbm.at[cid, pl.ds(sid * ZR, ZR)])


@functools.partial(
    pl.kernel,
    out_type=jax.ShapeDtypeStruct((NC, NPAD, 16), jnp.float32),
    mesh=_mesh,
    compiler_params=_sc_params,
    scratch_types=[
        pltpu.VMEM((C,), jnp.int32),
        pltpu.VMEM((C,), jnp.int32),
        pltpu.VMEM((C, 16), jnp.float32),
        pltpu.VMEM((C,), jnp.float32),
        pltpu.VMEM((C, 16), jnp.float32),
        pltpu.VMEM((C,), jnp.int32),
        pltpu.VMEM((C,), jnp.int32),
        pltpu.VMEM((C, 16), jnp.float32),
        pltpu.VMEM((C,), jnp.float32),
        pltpu.VMEM((C, 16), jnp.float32),
        pltpu.VMEM((TAIL,), jnp.int32),
        pltpu.VMEM((TAIL,), jnp.int32),
        pltpu.VMEM((TAIL, 16), jnp.float32),
        pltpu.VMEM((TAIL,), jnp.float32),
        pltpu.VMEM((TAIL, 16), jnp.float32),
        pltpu.VMEM((ZC, 16), jnp.float32),
        pltpu.VMEM_SHARED((NPAD,), jnp.float32),
        pltpu.VMEM_SHARED((NPAD, 16), jnp.float32),
        pltpu.SemaphoreType.DMA,
        pltpu.SemaphoreType.DMA,
        pltpu.SemaphoreType.DMA,
        pltpu.SemaphoreType.DMA,
        pltpu.SemaphoreType.DMA,
        pltpu.SemaphoreType.DMA,
    ],
)
def _edge_pass2(hrec_hbm, a2d_hbm, src_hbm, dst_hbm, out_hbm,
                src_v, dst_v, gbuf, ad_v, stag_v,
                src_b, dst_b, gbuf_b, ad_b, stag_b,
                src_t, dst_t, gbuf_t, ad_t, stag_t, zbuf, a2d_sh, acc_sh,
                sem, semia, semib, semga, semgb, semsc):
    cid = lax.axis_index("c")
    sid = lax.axis_index("s")
    wid = cid * NS + sid

    _fill_zero(zbuf)

    def zero_body(k, carry):
        pltpu.sync_copy(zbuf, acc_sh.at[pl.ds(sid * ZR + k * ZC, ZC)])
        return carry
    lax.fori_loop(0, ZR // ZC, zero_body, 0)

    @pl.when(sid == 0)
    def _load_a2d():
        pltpu.sync_copy(a2d_hbm, a2d_sh)

    plsc.subcore_barrier()

    ebase = wid * EPW
    iota = lax.iota(jnp.int32, L)
    col8 = jnp.full((L,), 8, jnp.int32)

    def compute(gb, adb, stbuf, csz):
        for v in range(csz // L):
            ad = adb[pl.ds(v * L, L)]
            row = iota + (v * L)
            a2s = plsc.load_gather(gb, [row, col8])
            p = _leaky_exp(a2s + ad)
            plsc.store_scatter(stbuf, [row, col8], p)
            for j in range(8):
                cj = jnp.full((L,), j, jnp.int32)
                hj = plsc.load_gather(gb, [row, cj])
                plsc.store_scatter(stbuf, [row, cj], hj * p)

    toff = ebase + NFULL * C
    pltpu.sync_copy(src_hbm.at[pl.ds(toff, TAIL)], src_t)
    pltpu.sync_copy(dst_hbm.at[pl.ds(toff, TAIL)], dst_t)
    pltpu.async_copy(hrec_hbm.at[src_t], gbuf_t, sem).wait()
    pltpu.async_copy(a2d_sh.at[dst_t], ad_t, sem).wait()
    compute(gbuf_t, ad_t, stag_t, TAIL)
    pltpu.sync_copy(stag_t, acc_sh.at[dst_t], add=True)

    def pair_body(k, carry):
        offa = ebase + (2 * k) * C
        offb = offa + C
        dia1 = pltpu.async_copy(src_hbm.at[pl.ds(offa, C)], src_v, semia)
        dia2 = pltpu.async_copy(dst_hbm.at[pl.ds(offa, C)], dst_v, semia)
        dib1 = pltpu.async_copy(src_hbm.at[pl.ds(offb, C)], src_b, semib)
        dib2 = pltpu.async_copy(dst_hbm.at[pl.ds(offb, C)], dst_b, semib)
        dia1.wait()
        dia2.wait()
        pltpu.async_copy(hrec_hbm.at[src_v], gbuf, semga).wait()
        pltpu.async_copy(a2d_sh.at[dst_v], ad_v, semga).wait()
        compute(gbuf, ad_v, stag_v, C)
        pltpu.sync_copy(stag_v, acc_sh.at[dst_v], add=True)
        dib1.wait()
        dib2.wait()
        pltpu.async_copy(hrec_hbm.at[src_b], gbuf_b, semgb).wait()
        pltpu.async_copy(a2d_sh.at[dst_b], ad_b, semgb).wait()
        compute(gbuf_b, ad_b, stag_b, C)
        pltpu.sync_copy(stag_b, acc_sh.at[dst_b], add=True)
        return carry
    lax.fori_loop(0, NFULL // 2, pair_body, 0)

    plsc.subcore_barrier()
    pltpu.sync_copy(acc_sh.at[pl.ds(sid * ZR, ZR)],
                    out_hbm.at[cid, pl.ds(sid * ZR, ZR)])


NB_A = 6256


def _tc_a_body(part_ref, w1_ref, b1_ref, w2_ref, as2_ref, ad2_ref,
               hrec_ref, a2d_ref):
    num = part_ref[0, :, 0:8] + part_ref[1, :, 0:8]
    den = part_ref[0, :, 8:16] + part_ref[1, :, 8:16]
    t = num / (den + 1e-16)
    ki = lax.broadcasted_iota(jnp.int32, (8, 64), 1) // 8
    hi = lax.broadcasted_iota(jnp.int32, (8, 64), 0)
    rep = (ki == hi).astype(jnp.float32)
    trep = jnp.dot(t, rep, preferred_element_type=jnp.float32)
    h1 = trep * w1_ref[:] + b1_ref[:]
    h1 = jnp.where(h1 > 0, h1, jnp.exp(h1) - 1.0)
    h2 = jnp.dot(h1, w2_ref[:], preferred_element_type=jnp.float32)
    a2s = jnp.sum(h2 * as2_ref[:], axis=1, keepdims=True)
    a2d = jnp.sum(h2 * ad2_ref[:], axis=1, keepdims=True)
    hrec_ref[:, 0:8] = h2
    hrec_ref[:, 8:9] = a2s
    hrec_ref[:, 9:16] = jnp.zeros((h2.shape[0], 7), jnp.float32)
    a2d_ref[:, :] = a2d


def _tc_a(part1, W1, b1, W2, as2, ad2):
    return pl.pallas_call(
        _tc_a_body,
        grid=(NPAD // NB_A,),
        in_specs=[
            pl.BlockSpec((NC, NB_A, 16), lambda i: (0, i, 0)),
            pl.BlockSpec((1, 64), lambda i: (0, 0)),
            pl.BlockSpec((1, 64), lambda i: (0, 0)),
            pl.BlockSpec((64, 8), lambda i: (0, 0)),
            pl.BlockSpec((1, 8), lambda i: (0, 0)),
            pl.BlockSpec((1, 8), lambda i: (0, 0)),
        ],
        out_specs=[
            pl.BlockSpec((NB_A, 16), lambda i: (i, 0)),
            pl.BlockSpec((NB_A, 1), lambda i: (i, 0)),
        ],
        out_shape=[
            jax.ShapeDtypeStruct((NPAD, 16), jnp.float32),
            jax.ShapeDtypeStruct((NPAD, 1), jnp.float32),
        ],
    )(part1, W1, b1, W2, as2, ad2)


NB_B = 6256


def _tc_b_body(part_ref, batch_ref, b2_ref, fcw_ref, fcb_ref, out_ref,
               sums, counts):
    i = pl.program_id(0)
    ng = pl.num_programs(0)

    @pl.when(i == 0)
    def _init():
        sums[:, :] = jnp.zeros((G, 8), jnp.float32)
        counts[:, :] = jnp.zeros((G, 1), jnp.float32)

    num = part_ref[0, :, 0:8] + part_ref[1, :, 0:8]
    den = part_ref[0, :, 8:9] + part_ref[1, :, 8:9]
    out2 = num / (den + 1e-16) + b2_ref[:]
    bb = batch_ref[:]
    gi = lax.broadcasted_iota(jnp.int32, (bb.shape[0], G), 1)
    onehot = (bb == gi).astype(jnp.float32)
    sums[:, :] += lax.dot_general(
        onehot, out2, (((0,), (0,)), ((), ())),
        preferred_element_type=jnp.float32)
    counts[:, :] += lax.dot_general(
        onehot, jnp.ones((bb.shape[0], 1), jnp.float32),
        (((0,), (0,)), ((), ())), preferred_element_type=jnp.float32)

    @pl.when(i == ng - 1)
    def _final():
        pooled = sums[:, :] / jnp.maximum(counts[:, :], 1.0)
        logit = jnp.dot(pooled, fcw_ref[:],
                        preferred_element_type=jnp.float32) + fcb_ref[:]
        out_ref[:] = jnp.squeeze(1.0 / (1.0 + jnp.exp(-logit)), axis=1)


def _tc_b(part2, batch2d, b2, fc_W, fc_b):
    return pl.pallas_call(
        _tc_b_body,
        grid=(NPAD // NB_B,),
        in_specs=[
            pl.BlockSpec((NC, NB_B, 16), lambda i: (0, i, 0)),
            pl.BlockSpec((NB_B, 1), lambda i: (i, 0)),
            pl.BlockSpec((1, 8), lambda i: (0, 0)),
            pl.BlockSpec((8, 1), lambda i: (0, 0)),
            pl.BlockSpec((1, 1), lambda i: (0, 0)),
        ],
        out_specs=pl.BlockSpec((G,), lambda i: (0,)),
        out_shape=jax.ShapeDtypeStruct((G,), jnp.float32),
        scratch_shapes=[
            pltpu.VMEM((G, 8), jnp.float32),
            pltpu.VMEM((G, 1), jnp.float32),
        ],
    )(part2, batch2d, b2, fc_W, fc_b)


def kernel(x, edge_index, batch, W1, att_src1, att_dst1, b1,
           W2, att_src2, att_dst2, b2, fc_W, fc_b):
    src = edge_index[0]
    dst = edge_index[1]
    xf = x.reshape(N)
    W1r = W1.reshape(8, 8)
    s1 = (W1r * att_src1).sum(1)
    d1 = (W1r * att_dst1).sum(1)
    coef = jnp.broadcast_to(
        jnp.concatenate([s1, d1])[:, None], (16, 16)).astype(jnp.float32)
    batch_p = jnp.concatenate(
        [batch, jnp.full((PAD,), G, jnp.int32)]).reshape(NPAD, 1)

    part1 = _edge_pass1(xf, src, dst, coef)
    hrec, a2d = _tc_a(part1, W1, b1.reshape(1, 64), W2, att_src2, att_dst2)
    part2 = _edge_pass2(hrec, a2d.reshape(NPAD), src, dst)
    return _tc_b(part2, batch_p, b2.reshape(1, 8),
                 fc_W, fc_b.reshape(1, 1))
